# baseline (speedup 1.0000x reference)
import jax
import jax.numpy as jnp
from jax.experimental import pallas as pl

_B, _H, _W, _D = 64, 24, 24, 768
_K = 9


def _conv_head_body(x_ref, dw_ref, pw_ref, b_ref, o_ref):
    x = x_ref[0]
    zc = jnp.zeros((_H, 1, _D), jnp.float32)
    xp = jnp.concatenate([zc, x, zc], axis=1)
    zr = jnp.zeros((1, _W + 2, _D), jnp.float32)
    xp = jnp.concatenate([zr, xp, zr], axis=0)
    acc = xp[0:_H, 0:_W, :] * dw_ref[0, :]
    for t in range(1, 9):
        i, j = divmod(t, 3)
        acc = acc + xp[i:i + _H, j:j + _W, :] * dw_ref[t, :]
    h = acc.reshape(_H * _W, _D)
    logits = jnp.dot(h, pw_ref[...], preferred_element_type=jnp.float32)
    logits = logits + b_ref[0:1, :_K]
    m = jnp.max(logits, axis=-1, keepdims=True)
    e = jnp.exp(logits - m)
    s = jnp.sum(e, axis=-1, keepdims=True)
    o_ref[0] = (e / s).reshape(_H, _W, _K)


def kernel(tok2d, dw_w, pw_w, pw_b):
    dwt = jnp.transpose(dw_w.reshape(_D, 9), (1, 0))
    dwt = jnp.pad(dwt, ((0, 7), (0, 0)))
    pwt = jnp.transpose(pw_w.reshape(_K, _D), (1, 0))
    bias = jnp.zeros((8, 16), jnp.float32).at[0, :_K].set(pw_b)
    out = pl.pallas_call(
        _conv_head_body,
        grid=(_B,),
        in_specs=[
            pl.BlockSpec((1, _H, _W, _D), lambda b: (b, 0, 0, 0)),
            pl.BlockSpec((16, _D), lambda b: (0, 0)),
            pl.BlockSpec((_D, _K), lambda b: (0, 0)),
            pl.BlockSpec((8, 16), lambda b: (0, 0)),
        ],
        out_specs=pl.BlockSpec((1, _H, _W, _K), lambda b: (b, 0, 0, 0)),
        out_shape=jax.ShapeDtypeStruct((_B, _H, _W, _K), jnp.float32),
    )(tok2d, dwt, pwt, bias)
    return out


# speedup vs baseline: 1.7189x; 1.7189x over previous
import jax
import jax.numpy as jnp
from jax.experimental import pallas as pl

_B, _H, _W, _D = 64, 24, 24, 768
_K = 9


def _conv_head_body(x_ref, w_ref, b_ref, o_ref):
    x = x_ref[0].reshape(_H * _W, _D)
    z = jnp.dot(x, w_ref[...], preferred_element_type=jnp.float32)
    z = z.reshape(_H, _W, 128)
    zc = jnp.zeros((_H, 1, 128), jnp.float32)
    zp = jnp.concatenate([zc, z, zc], axis=1)
    zr = jnp.zeros((1, _W + 2, 128), jnp.float32)
    zp = jnp.concatenate([zr, zp, zr], axis=0)
    acc = jax.lax.slice(zp, (0, 0, 0), (_H, _W, _K))
    for t in range(1, 9):
        i, j = divmod(t, 3)
        acc = acc + jax.lax.slice(zp, (i, j, _K * t), (i + _H, j + _W, _K * t + _K))
    logits = acc + b_ref[0:1, 0:1, :]
    m = jnp.max(logits, axis=-1, keepdims=True)
    e = jnp.exp(logits - m)
    s = jnp.sum(e, axis=-1, keepdims=True)
    o_ref[0] = e / s


def kernel(tok2d, dw_w, pw_w, pw_b):
    dwt = dw_w.reshape(_D, 9)
    pwm = jnp.transpose(pw_w.reshape(_K, _D))
    wall = (dwt[:, :, None] * pwm[:, None, :]).reshape(_D, 81)
    wall = jnp.pad(wall, ((0, 0), (0, 47)))
    bias = jnp.zeros((1, 1, _K), jnp.float32).at[0, 0, :].set(pw_b)
    out = pl.pallas_call(
        _conv_head_body,
        grid=(_B,),
        in_specs=[
            pl.BlockSpec((1, _H, _W, _D), lambda b: (b, 0, 0, 0)),
            pl.BlockSpec((_D, 128), lambda b: (0, 0)),
            pl.BlockSpec((1, 1, _K), lambda b: (0, 0, 0)),
        ],
        out_specs=pl.BlockSpec((1, _H, _W, _K), lambda b: (b, 0, 0, 0)),
        out_shape=jax.ShapeDtypeStruct((_B, _H, _W, _K), jnp.float32),
    )(tok2d, wall, bias)
    return out


# speedup vs baseline: 2.3168x; 1.3479x over previous
import jax
import jax.numpy as jnp
from jax.experimental import pallas as pl

_B, _H, _W, _D = 64, 24, 24, 768
_K = 9


_BB = 4


def _conv_head_body(x_ref, w_ref, b_ref, o_ref):
    x = x_ref[...].reshape(_BB * _H * _W, _D)
    z = jnp.dot(x, w_ref[...], preferred_element_type=jnp.float32)
    z = z.reshape(_BB, _H, _W, 128)
    zc = jnp.zeros((_BB, _H, 1, 128), jnp.float32)
    zp = jnp.concatenate([zc, z, zc], axis=2)
    zr = jnp.zeros((_BB, 1, _W + 2, 128), jnp.float32)
    zp = jnp.concatenate([zr, zp, zr], axis=1)
    acc = jax.lax.slice(zp, (0, 0, 0, 0), (_BB, _H, _W, _K))
    for t in range(1, 9):
        i, j = divmod(t, 3)
        acc = acc + jax.lax.slice(
            zp, (0, i, j, _K * t), (_BB, i + _H, j + _W, _K * t + _K))
    logits = acc + b_ref[0:1, 0:1, 0:1, :]
    m = jnp.max(logits, axis=-1, keepdims=True)
    e = jnp.exp(logits - m)
    s = jnp.sum(e, axis=-1, keepdims=True)
    o_ref[...] = e / s


def kernel(tok2d, dw_w, pw_w, pw_b):
    dwt = dw_w.reshape(_D, 9)
    pwm = jnp.transpose(pw_w.reshape(_K, _D))
    wall = (dwt[:, :, None] * pwm[:, None, :]).reshape(_D, 81)
    wall = jnp.pad(wall, ((0, 0), (0, 47)))
    bias = jnp.zeros((1, 1, 1, _K), jnp.float32).at[0, 0, 0, :].set(pw_b)
    out = pl.pallas_call(
        _conv_head_body,
        grid=(_B // _BB,),
        in_specs=[
            pl.BlockSpec((_BB, _H, _W, _D), lambda b: (b, 0, 0, 0)),
            pl.BlockSpec((_D, 128), lambda b: (0, 0)),
            pl.BlockSpec((1, 1, 1, _K), lambda b: (0, 0, 0, 0)),
        ],
        out_specs=pl.BlockSpec((_BB, _H, _W, _K), lambda b: (b, 0, 0, 0)),
        out_shape=jax.ShapeDtypeStruct((_B, _H, _W, _K), jnp.float32),
    )(tok2d, wall, bias)
    return out


# speedup vs baseline: 2.7311x; 1.1788x over previous
import jax
import jax.numpy as jnp
from jax.experimental import pallas as pl
from jax.experimental.pallas import tpu as pltpu

_B, _H, _W, _D = 64, 24, 24, 768
_K = 9
_BB = 4


def _conv_head_body(x_ref, w_ref, t_ref, b_ref, o_ref):
    x = x_ref[...].reshape(_BB * _H * _W, _D)
    z = jnp.dot(x, w_ref[...], preferred_element_type=jnp.float32)
    z = z.reshape(_BB, _H, _W, 128)
    wio = jax.lax.broadcasted_iota(jnp.int32, (_BB, _H, _W, 128), 2)
    pm = jnp.where(wio == 0, 0.0, pltpu.roll(z, 1, axis=2))
    pp = jnp.where(wio == _W - 1, 0.0, pltpu.roll(z, _W - 1, axis=2))
    zrow = jnp.zeros((_BB, 1, _W, 128), jnp.float32)
    pj = [jnp.concatenate([zrow, p, zrow], axis=1) for p in (pm, z, pp)]
    terms = []
    for t in range(9):
        i, j = divmod(t, 3)
        terms.append(jax.lax.slice(
            pj[j], (0, i, 0, 0), (_BB, i + _H, _W, 128)))
    cio = jax.lax.broadcasted_iota(jnp.int32, (_BB, _H, _W, 128), 3)
    s = terms[8]
    for t in range(7, -1, -1):
        s = jnp.where(cio < _K * (t + 1), terms[t], s)
    s = s.reshape(_BB * _H * _W, 128)
    acc = jnp.dot(s, t_ref[...], preferred_element_type=jnp.float32)
    logits = jax.lax.slice(acc, (0, 0), (_BB * _H * _W, _K)) + b_ref[0, 0]
    m = jnp.max(logits, axis=-1, keepdims=True)
    e = jnp.exp(logits - m)
    den = jnp.sum(e, axis=-1, keepdims=True)
    o_ref[...] = (e / den).reshape(_BB, _H, _W, _K)


def kernel(tok2d, dw_w, pw_w, pw_b):
    dwt = dw_w.reshape(_D, 9)
    pwm = jnp.transpose(pw_w.reshape(_K, _D))
    wall = (dwt[:, :, None] * pwm[:, None, :]).reshape(_D, 81)
    wall = jnp.pad(wall, ((0, 0), (0, 47)))
    rows = jnp.arange(81)
    tmat = jnp.zeros((128, 128), jnp.float32).at[rows, rows % _K].set(1.0)
    bias = jnp.zeros((1, 1, 1, _K), jnp.float32).at[0, 0, 0, :].set(pw_b)
    out = pl.pallas_call(
        _conv_head_body,
        grid=(_B // _BB,),
        in_specs=[
            pl.BlockSpec((_BB, _H, _W, _D), lambda b: (b, 0, 0, 0)),
            pl.BlockSpec((_D, 128), lambda b: (0, 0)),
            pl.BlockSpec((128, 128), lambda b: (0, 0)),
            pl.BlockSpec((1, 1, 1, _K), lambda b: (0, 0, 0, 0)),
        ],
        out_specs=pl.BlockSpec((_BB, _H, _W, _K), lambda b: (b, 0, 0, 0)),
        out_shape=jax.ShapeDtypeStruct((_B, _H, _W, _K), jnp.float32),
    )(tok2d, wall, tmat, bias)
    return out


# speedup vs baseline: 2.8283x; 1.0356x over previous
import jax
import jax.numpy as jnp
from jax.experimental import pallas as pl
from jax.experimental.pallas import tpu as pltpu

_B, _H, _W, _D = 64, 24, 24, 768
_K = 9
_BB = 8


def _conv_head_body(x_ref, w_ref, t_ref, b_ref, o_ref):
    x = x_ref[...].reshape(_BB * _H * _W, _D)
    z = jnp.dot(x, w_ref[...], preferred_element_type=jnp.float32)
    z = z.reshape(_BB, _H, _W, 128)
    wio = jax.lax.broadcasted_iota(jnp.int32, (_BB, _H, _W, 128), 2)
    pm = jnp.where(wio == 0, 0.0, pltpu.roll(z, 1, axis=2))
    pp = jnp.where(wio == _W - 1, 0.0, pltpu.roll(z, _W - 1, axis=2))
    zrow = jnp.zeros((_BB, 1, _W, 128), jnp.float32)
    pj = [jnp.concatenate([zrow, p, zrow], axis=1) for p in (pm, z, pp)]
    terms = []
    for t in range(9):
        i, j = divmod(t, 3)
        terms.append(jax.lax.slice(
            pj[j], (0, i, 0, 0), (_BB, i + _H, _W, 128)))
    cio = jax.lax.broadcasted_iota(jnp.int32, (_BB, _H, _W, 128), 3)
    s = terms[8]
    for t in range(7, -1, -1):
        s = jnp.where(cio < _K * (t + 1), terms[t], s)
    s = s.reshape(_BB * _H * _W, 128)
    acc = jnp.dot(s, t_ref[...], preferred_element_type=jnp.float32)
    logits = jax.lax.slice(acc, (0, 0), (_BB * _H * _W, _K)) + b_ref[0, 0]
    m = jnp.max(logits, axis=-1, keepdims=True)
    e = jnp.exp(logits - m)
    den = jnp.sum(e, axis=-1, keepdims=True)
    o_ref[...] = (e / den).reshape(_BB, _H, _W, _K)


def kernel(tok2d, dw_w, pw_w, pw_b):
    dwt = dw_w.reshape(_D, 9)
    pwm = jnp.transpose(pw_w.reshape(_K, _D))
    wall = (dwt[:, :, None] * pwm[:, None, :]).reshape(_D, 81)
    wall = jnp.pad(wall, ((0, 0), (0, 47)))
    rows = jnp.arange(81)
    tmat = jnp.zeros((128, 128), jnp.float32).at[rows, rows % _K].set(1.0)
    bias = jnp.zeros((1, 1, 1, _K), jnp.float32).at[0, 0, 0, :].set(pw_b)
    out = pl.pallas_call(
        _conv_head_body,
        grid=(_B // _BB,),
        in_specs=[
            pl.BlockSpec((_BB, _H, _W, _D), lambda b: (b, 0, 0, 0)),
            pl.BlockSpec((_D, 128), lambda b: (0, 0)),
            pl.BlockSpec((128, 128), lambda b: (0, 0)),
            pl.BlockSpec((1, 1, 1, _K), lambda b: (0, 0, 0, 0)),
        ],
        out_specs=pl.BlockSpec((_BB, _H, _W, _K), lambda b: (b, 0, 0, 0)),
        out_shape=jax.ShapeDtypeStruct((_B, _H, _W, _K), jnp.float32),
    )(tok2d, wall, tmat, bias)
    return out
